# baseline (speedup 1.0000x reference)
import functools

import jax
import jax.numpy as jnp
from jax import lax
from jax.experimental import pallas as pl
from jax.experimental.pallas import tpu as pltpu
from jax.experimental.pallas import tpu_sc as plsc

_NBUF = 4


@functools.lru_cache(maxsize=None)
def _build_gather(N, V, D, NC, NS, C):
    NW = NC * NS
    b_per_w = N // NW
    n_chunks = b_per_w // C
    n_grp = n_chunks // _NBUF

    stripe = V // NS

    mesh = plsc.VectorSubcoreMesh(core_axis_name="c", subcore_axis_name="s")

    @functools.partial(
        pl.kernel,
        mesh=mesh,
        out_type=jax.ShapeDtypeStruct((NW, n_chunks, C, D), jnp.float32),
        scratch_types=[
            pltpu.VMEM((n_chunks, C), jnp.int32),
            pltpu.VMEM((_NBUF, C, D), jnp.float32),
            pltpu.VMEM_SHARED((V, D), jnp.float32),
        ]
        + [pltpu.SemaphoreType.DMA] * (2 * _NBUF),
    )
    def k(table_hbm, idx_hbm, out_hbm, idx_v, rows_v, table_sh, *sems):
        gsems = sems[:_NBUF]
        ssems = sems[_NBUF:]
        cid = lax.axis_index("c")
        sid = lax.axis_index("s")
        wid = sid * NC + cid

        off = sid * stripe
        pltpu.sync_copy(table_hbm.at[pl.ds(off, stripe)], table_sh.at[pl.ds(off, stripe)])

        head = 8
        pltpu.sync_copy(idx_hbm.at[wid, pl.ds(0, head)], idx_v.at[pl.ds(0, head)])
        plsc.subcore_barrier()

        def gcopy(j, b):
            return pltpu.make_async_copy(
                table_sh.at[idx_v.at[j]], rows_v.at[b], gsems[b]
            )

        def scopy(j, b):
            return pltpu.make_async_copy(rows_v.at[b], out_hbm.at[wid, j], ssems[b])

        for b in range(_NBUF - 1):
            gcopy(b, b).start()

        pltpu.sync_copy(
            idx_hbm.at[wid, pl.ds(head, n_chunks - head)],
            idx_v.at[pl.ds(head, n_chunks - head)],
        )

        gcopy(0, 0).wait()
        scopy(0, 0).start()
        gcopy(3, 3).start()
        for b in range(1, _NBUF):
            gcopy(b, b).wait()
            scopy(b, b).start()
            scopy(b - 1, b - 1).wait()
            gcopy(b + _NBUF - 1, b - 1).start()

        def grp(g, carry):
            for b in range(_NBUF):
                j = _NBUF * g + b
                gcopy(j, b).wait()
                scopy(j, b).start()
                scopy(j - 1, (b - 1) % _NBUF).wait()
                gcopy(j + _NBUF - 1, (b - 1) % _NBUF).start()
            return carry

        lax.fori_loop(1, n_grp - 1, grp, 0)

        j0 = _NBUF * (n_grp - 1)
        gcopy(j0, 0).wait()
        scopy(j0, 0).start()
        scopy(j0 - 1, _NBUF - 1).wait()
        gcopy(j0 + _NBUF - 1, _NBUF - 1).start()
        for b in range(1, _NBUF):
            j = j0 + b
            gcopy(j, b).wait()
            scopy(j, b).start()
            scopy(j - 1, b - 1).wait()
        scopy(n_chunks - 1, _NBUF - 1).wait()

    return k


def kernel(x, doy_table, month_table):
    B, H = x.shape
    V, D = doy_table.shape
    N = B * H
    info = plsc.get_sparse_core_info()
    NC, NS = info.num_cores, info.num_subcores
    NW = NC * NS
    C = 128
    Vp = -(-V // (8 * NS)) * (8 * NS)
    table_p = jnp.pad(doy_table, ((0, Vp - V), (0, 0)))
    xw = x.reshape(NW, (N // NW) // C, C).astype(jnp.int32)
    out = _build_gather(N, Vp, D, NC, NS, C)(table_p, xw)
    return out.reshape(B, H, D)


# speedup vs baseline: 1.0013x; 1.0013x over previous
import functools

import jax
import jax.numpy as jnp
from jax import lax
from jax.experimental import pallas as pl
from jax.experimental.pallas import tpu as pltpu
from jax.experimental.pallas import tpu_sc as plsc

_NBUF = 4


@functools.lru_cache(maxsize=None)
def _build_gather(N, V, D, NC, NS, C):
    NW = NC * NS
    b_per_w = N // NW
    n_chunks = b_per_w // C
    n_grp = n_chunks // _NBUF

    stripe = V // NS

    mesh = plsc.VectorSubcoreMesh(core_axis_name="c", subcore_axis_name="s")

    @functools.partial(
        pl.kernel,
        mesh=mesh,
        out_type=jax.ShapeDtypeStruct((NW, n_chunks, C, D), jnp.float32),
        scratch_types=[
            pltpu.VMEM((n_chunks, C), jnp.int32),
            pltpu.VMEM((_NBUF, C, D), jnp.float32),
            pltpu.VMEM_SHARED((V, D), jnp.float32),
        ]
        + [pltpu.SemaphoreType.DMA] * (2 * _NBUF),
    )
    def k(table_hbm, idx_hbm, out_hbm, idx_v, rows_v, table_sh, *sems):
        gsems = sems[:_NBUF]
        ssems = sems[_NBUF:]
        cid = lax.axis_index("c")
        sid = lax.axis_index("s")
        wid = sid * NC + cid

        off = sid * stripe
        pltpu.sync_copy(table_hbm.at[pl.ds(off, stripe)], table_sh.at[pl.ds(off, stripe)])

        head = 8
        pltpu.sync_copy(idx_hbm.at[wid, pl.ds(0, head)], idx_v.at[pl.ds(0, head)])
        plsc.subcore_barrier()

        for b in range(_NBUF):
            pltpu.async_copy(table_sh.at[idx_v.at[b]], rows_v.at[b], gsems[b])

        pltpu.sync_copy(
            idx_hbm.at[wid, pl.ds(head, n_chunks - head)],
            idx_v.at[pl.ds(head, n_chunks - head)],
        )

        def pstore(p, h):
            return pltpu.make_async_copy(
                rows_v.at[pl.ds(2 * h, 2)],
                out_hbm.at[wid, pl.ds(2 * p, 2)],
                ssems[h],
            )

        def grp(g, carry):
            for h in range(2):
                p = 2 * g + h
                j = 2 * p
                pltpu.make_async_copy(
                    table_sh.at[idx_v.at[j]], rows_v.at[2 * h], gsems[2 * h]
                ).wait()
                pltpu.make_async_copy(
                    table_sh.at[idx_v.at[j + 1]], rows_v.at[2 * h + 1], gsems[2 * h + 1]
                ).wait()
                pstore(p, h).start()
                pstore(p, h).wait()
                pltpu.async_copy(
                    table_sh.at[idx_v.at[j + _NBUF]], rows_v.at[2 * h], gsems[2 * h]
                )
                pltpu.async_copy(
                    table_sh.at[idx_v.at[j + _NBUF + 1]],
                    rows_v.at[2 * h + 1],
                    gsems[2 * h + 1],
                )
            return carry

        lax.fori_loop(0, n_grp - 1, grp, 0)

        for h in range(2):
            p = 2 * (n_grp - 1) + h
            j = 2 * p
            pltpu.make_async_copy(
                table_sh.at[idx_v.at[j]], rows_v.at[2 * h], gsems[2 * h]
            ).wait()
            pltpu.make_async_copy(
                table_sh.at[idx_v.at[j + 1]], rows_v.at[2 * h + 1], gsems[2 * h + 1]
            ).wait()
            pstore(p, h).start()
        for h in range(2):
            pstore(2 * (n_grp - 1) + h, h).wait()

    return k


def kernel(x, doy_table, month_table):
    B, H = x.shape
    V, D = doy_table.shape
    N = B * H
    info = plsc.get_sparse_core_info()
    NC, NS = info.num_cores, info.num_subcores
    NW = NC * NS
    C = 128
    Vp = -(-V // (8 * NS)) * (8 * NS)
    table_p = jnp.pad(doy_table, ((0, Vp - V), (0, 0)))
    xw = x.reshape(NW, (N // NW) // C, C).astype(jnp.int32)
    out = _build_gather(N, Vp, D, NC, NS, C)(table_p, xw)
    return out.reshape(B, H, D)


# speedup vs baseline: 1.0099x; 1.0085x over previous
import functools

import jax
import jax.numpy as jnp
from jax import lax
from jax.experimental import pallas as pl
from jax.experimental.pallas import tpu as pltpu
from jax.experimental.pallas import tpu_sc as plsc

_NBUF = 4


@functools.lru_cache(maxsize=None)
def _build_gather(N, V, D, NC, NS, C):
    NW = NC * NS
    b_per_w = N // NW
    n_chunks = b_per_w // C
    n_grp = n_chunks // _NBUF

    stripe = V // NS

    mesh = plsc.VectorSubcoreMesh(core_axis_name="c", subcore_axis_name="s")

    @functools.partial(
        pl.kernel,
        mesh=mesh,
        out_type=jax.ShapeDtypeStruct((NW, n_chunks, C, D), jnp.float32),
        scratch_types=[
            pltpu.VMEM((n_chunks, C), jnp.int32),
            pltpu.VMEM((_NBUF, C, D), jnp.float32),
            pltpu.VMEM_SHARED((V, D), jnp.float32),
        ]
        + [pltpu.SemaphoreType.DMA] * (2 * _NBUF),
    )
    def k(table_hbm, idx_hbm, out_hbm, idx_v, rows_v, table_sh, *sems):
        gsems = sems[:_NBUF]
        ssems = sems[_NBUF:]
        cid = lax.axis_index("c")
        sid = lax.axis_index("s")
        wid = sid * NC + cid

        off = sid * stripe
        pltpu.sync_copy(table_hbm.at[pl.ds(off, stripe)], table_sh.at[pl.ds(off, stripe)])

        head = 8
        pltpu.sync_copy(idx_hbm.at[wid, pl.ds(0, head)], idx_v.at[pl.ds(0, head)])
        plsc.subcore_barrier()

        for b in range(_NBUF):
            pltpu.async_copy(table_sh.at[idx_v.at[b]], rows_v.at[b], gsems[b])

        pltpu.sync_copy(
            idx_hbm.at[wid, pl.ds(head, n_chunks - head)],
            idx_v.at[pl.ds(head, n_chunks - head)],
        )

        def grp(g, carry):
            for b in range(_NBUF):
                j = _NBUF * g + b
                pltpu.make_async_copy(
                    table_sh.at[idx_v.at[j]], rows_v.at[b], gsems[b]
                ).wait()
                pltpu.async_copy(rows_v.at[b], out_hbm.at[wid, j], ssems[b])
                pltpu.make_async_copy(
                    rows_v.at[b], out_hbm.at[wid, j], ssems[b]
                ).wait()
                pltpu.async_copy(
                    table_sh.at[idx_v.at[j + _NBUF]], rows_v.at[b], gsems[b]
                )
            return carry

        lax.fori_loop(0, n_grp - 1, grp, 0)

        for b in range(_NBUF):
            j = _NBUF * (n_grp - 1) + b
            pltpu.make_async_copy(
                table_sh.at[idx_v.at[j]], rows_v.at[b], gsems[b]
            ).wait()
            pltpu.async_copy(rows_v.at[b], out_hbm.at[wid, j], ssems[b])
        for b in range(_NBUF):
            j = _NBUF * (n_grp - 1) + b
            pltpu.make_async_copy(rows_v.at[b], out_hbm.at[wid, j], ssems[b]).wait()

    return k


def kernel(x, doy_table, month_table):
    B, H = x.shape
    V, D = doy_table.shape
    N = B * H
    info = plsc.get_sparse_core_info()
    NC, NS = info.num_cores, info.num_subcores
    NW = NC * NS
    C = 128
    Vp = -(-V // (8 * NS)) * (8 * NS)
    table_p = jnp.pad(doy_table, ((0, Vp - V), (0, 0)))
    xw = x.reshape(NW, (N // NW) // C, C).astype(jnp.int32)
    out = _build_gather(N, Vp, D, NC, NS, C)(table_p, xw)
    return out.reshape(B, H, D)
